# baseline (speedup 1.0000x reference)
import functools

import jax
import jax.numpy as jnp
from jax import lax
from jax.experimental import pallas as pl
from jax.experimental.pallas import tpu as pltpu
from jax.experimental.pallas import tpu_sc as plsc

_SEQ = 8192
_DIM = 1024
_BATCH = 4
_NC = 2
_NS = 16
_NW = _NC * _NS
_ROWS_PER_W = _SEQ // _NW
_CHUNK_ROWS = 32
_CHUNK = _CHUNK_ROWS * _DIM
_N_CHUNKS = _ROWS_PER_W // _CHUNK_ROWS
_UNROLL = 8

_mesh = plsc.VectorSubcoreMesh(core_axis_name="c", subcore_axis_name="s")


@functools.partial(
    pl.kernel,
    mesh=_mesh,
    out_type=jax.ShapeDtypeStruct((_BATCH * _SEQ * _DIM,), jnp.float32),
    scratch_types=[
        pltpu.VMEM((_CHUNK,), jnp.float32),
        pltpu.VMEM((_CHUNK,), jnp.float32),
    ],
)
def _sc_add(x_hbm, pos_hbm, out_hbm, pos_v, x_v):
    wid = lax.axis_index("s") * _NC + lax.axis_index("c")

    def chunk_body(c, carry):
        seq_off = (wid * _ROWS_PER_W + c * _CHUNK_ROWS) * _DIM
        pltpu.sync_copy(pos_hbm.at[pl.ds(seq_off, _CHUNK)], pos_v)

        def batch_body(b, carry2):
            off = b * (_SEQ * _DIM) + seq_off
            pltpu.sync_copy(x_hbm.at[pl.ds(off, _CHUNK)], x_v)

            def add_body(i, carry3):
                for u in range(_UNROLL):
                    s = pl.ds((i * _UNROLL + u) * 16, 16)
                    plsc.addupdate(x_v.at[s], pos_v[s])
                return carry3

            lax.fori_loop(0, _CHUNK // (16 * _UNROLL), add_body, 0)
            pltpu.sync_copy(x_v, out_hbm.at[pl.ds(off, _CHUNK)])
            return carry2

        lax.fori_loop(0, _BATCH, batch_body, 0)
        return carry

    lax.fori_loop(0, _N_CHUNKS, chunk_body, 0)


def kernel(x, pos_table):
    out = _sc_add(x.reshape(-1), pos_table.reshape(-1))
    return out.reshape(x.shape)


# speedup vs baseline: 1.2174x; 1.2174x over previous
import functools

import jax
import jax.numpy as jnp
from jax import lax
from jax.experimental import pallas as pl
from jax.experimental.pallas import tpu as pltpu
from jax.experimental.pallas import tpu_sc as plsc

_SEQ = 8192
_DIM = 1024
_BATCH = 4
_NC = 2
_NS = 16
_NW = _NC * _NS
_ROWS_PER_W = _SEQ // _NW
_CHUNK_ROWS = 16
_CHUNK = _CHUNK_ROWS * _DIM
_N_CHUNKS = _ROWS_PER_W // _CHUNK_ROWS
_UNROLL = 8

_mesh = plsc.VectorSubcoreMesh(core_axis_name="c", subcore_axis_name="s")


@functools.partial(
    pl.kernel,
    mesh=_mesh,
    out_type=jax.ShapeDtypeStruct((_BATCH * _SEQ * _DIM,), jnp.float32),
    scratch_types=(
        [pltpu.VMEM((_CHUNK,), jnp.float32)] * 5
        + [pltpu.SemaphoreType.DMA] * 8
    ),
)
def _sc_add(x_hbm, pos_hbm, out_hbm, pos_v, xb0, xb1, xb2, xb3,
            is0, is1, is2, is3, os0, os1, os2, os3):
    xb = [xb0, xb1, xb2, xb3]
    in_sem = [is0, is1, is2, is3]
    out_sem = [os0, os1, os2, os3]
    wid = lax.axis_index("s") * _NC + lax.axis_index("c")
    row0 = wid * _ROWS_PER_W

    def x_off(c, b):
        return b * (_SEQ * _DIM) + (row0 + c * _CHUNK_ROWS) * _DIM

    def start_in(c, b):
        pltpu.make_async_copy(
            x_hbm.at[pl.ds(x_off(c, b), _CHUNK)], xb[b], in_sem[b]
        ).start()

    def wait_in(b):
        pltpu.make_async_copy(
            x_hbm.at[pl.ds(0, _CHUNK)], xb[b], in_sem[b]
        ).wait()

    def start_out(c, b):
        pltpu.make_async_copy(
            xb[b], out_hbm.at[pl.ds(x_off(c, b), _CHUNK)], out_sem[b]
        ).start()

    def wait_out(b):
        pltpu.make_async_copy(
            xb[b], out_hbm.at[pl.ds(0, _CHUNK)], out_sem[b]
        ).wait()

    def compute(b):
        def body(i, carry):
            for u in range(_UNROLL):
                s = pl.ds((i * _UNROLL + u) * 16, 16)
                plsc.addupdate(xb[b].at[s], pos_v[s])
            return carry

        lax.fori_loop(0, _CHUNK // (16 * _UNROLL), body, 0)

    def load_pos(c):
        pltpu.sync_copy(
            pos_hbm.at[pl.ds((row0 + c * _CHUNK_ROWS) * _DIM, _CHUNK)], pos_v
        )

    def chunk_body(c, first, last):
        load_pos(c)
        for b in range(4):
            nb = (b + 2) % 4
            wait_in(b)
            if not (first and b < 2):
                wait_out(nb)
            if not (last and b >= 2):
                start_in(c + (1 if b >= 2 else 0), nb)
            compute(b)
            start_out(c, b)

    start_in(0, 0)
    start_in(0, 1)
    chunk_body(0, first=True, last=(_N_CHUNKS == 1))

    def fori_body(c, carry):
        chunk_body(c, first=False, last=False)
        return carry

    lax.fori_loop(1, _N_CHUNKS - 1, fori_body, 0)
    chunk_body(_N_CHUNKS - 1, first=False, last=True)
    wait_out(2)
    wait_out(3)


def kernel(x, pos_table):
    out = _sc_add(x.reshape(-1), pos_table.reshape(-1))
    return out.reshape(x.shape)


# speedup vs baseline: 5.1717x; 4.2480x over previous
import jax
import jax.numpy as jnp
from jax.experimental import pallas as pl


_SEQ_BLOCK = 2048


def _add_body(x_ref, pos_ref, out_ref):
    out_ref[...] = x_ref[...] + pos_ref[...]


def kernel(x, pos_table):
    batch, seq, dim = x.shape
    grid = (seq // _SEQ_BLOCK, batch)
    return pl.pallas_call(
        _add_body,
        grid=grid,
        in_specs=[
            pl.BlockSpec((1, _SEQ_BLOCK, dim), lambda s, b: (b, s, 0)),
            pl.BlockSpec((_SEQ_BLOCK, dim), lambda s, b: (s, 0)),
        ],
        out_specs=pl.BlockSpec((1, _SEQ_BLOCK, dim), lambda s, b: (b, s, 0)),
        out_shape=jax.ShapeDtypeStruct(x.shape, x.dtype),
    )(x, pos_table)
